# baseline (speedup 1.0000x reference)
import functools

import jax
import jax.numpy as jnp
from jax import lax
from jax.experimental import pallas as pl
from jax.experimental.pallas import tpu as pltpu
from jax.experimental.pallas import tpu_sc as plsc

MAX_LEN = 4096
D_MODEL = 768
BATCH = 4
NCOND = 3

NC, NS, NLANES = 2, 16, 16
NW = NC * NS
ROWS_PER_W = MAX_LEN // NW
CHUNK = 64
REP = 256
_sc_mesh = plsc.VectorSubcoreMesh(core_axis_name="c", subcore_axis_name="s")


D_PACK = D_MODEL // 2


@functools.partial(
    pl.kernel,
    out_type=jax.ShapeDtypeStruct((MAX_LEN, D_PACK), jnp.int32),
    mesh=_sc_mesh,
    scratch_types=[
        pltpu.VMEM((CHUNK,), jnp.int32),
        pltpu.VMEM((CHUNK,), jnp.int32),
        pltpu.VMEM((CHUNK, D_PACK), jnp.int32),
        pltpu.VMEM((CHUNK, D_PACK), jnp.int32),
        pltpu.SemaphoreType.DMA,
        pltpu.SemaphoreType.DMA,
    ],
)
def _sc_cond_emb(idx_hbm, ctab_hbm, cemb_hbm, idx0, idx1, buf0, buf1, sem0, sem1):
    wid = lax.axis_index("s") * NC + lax.axis_index("c")
    base = wid * ROWS_PER_W
    pltpu.sync_copy(idx_hbm.at[pl.ds(base, CHUNK)], idx0)
    pltpu.sync_copy(idx_hbm.at[pl.ds(base + CHUNK, CHUNK)], idx1)
    lanes = lax.broadcasted_iota(jnp.int32, (NLANES,), 0)
    for t in range(CHUNK // NLANES):
        o0 = ((base + t * NLANES) & (REP - 1)) + lanes
        idx0[pl.ds(t * NLANES, NLANES)] += NCOND * (o0 & (REP - 1))
        o1 = ((base + CHUNK + t * NLANES) & (REP - 1)) + lanes
        idx1[pl.ds(t * NLANES, NLANES)] += NCOND * (o1 & (REP - 1))
    cp0 = pltpu.async_copy(ctab_hbm.at[idx0], buf0, sem0)
    cp1 = pltpu.async_copy(ctab_hbm.at[idx1], buf1, sem1)
    cp0.wait()
    pltpu.sync_copy(buf0, cemb_hbm.at[pl.ds(base, CHUNK)])
    cp1.wait()
    pltpu.sync_copy(buf1, cemb_hbm.at[pl.ds(base + CHUNK, CHUNK)])


BL = 256
NB = MAX_LEN // BL


def _dense_body(in_ref, pos_ref, cemb_ref, out_ref):
    x = cemb_ref[...]
    lo = lax.bitcast_convert_type(lax.shift_left(x, 16), jnp.float32)
    hi = lax.bitcast_convert_type(lax.bitwise_and(x, jnp.int32(-65536)), jnp.float32)
    add_lo = pos_ref[:, :D_PACK] + lo
    add_hi = pos_ref[:, D_PACK:] + hi
    out_ref[:, :, :D_PACK] = in_ref[:, :, :D_PACK] + add_lo[None, :, :]
    out_ref[:, :, D_PACK:] = in_ref[:, :, D_PACK:] + add_hi[None, :, :]


@jax.jit
def _dense_add(inputs, pos_table, cond_emb):
    return pl.pallas_call(
        _dense_body,
        grid=(NB,),
        in_specs=[
            pl.BlockSpec((BATCH, BL, D_MODEL), lambda i: (0, i, 0)),
            pl.BlockSpec((BL, D_MODEL), lambda i: (i, 0)),
            pl.BlockSpec((BL, D_PACK), lambda i: (i, 0)),
        ],
        out_specs=pl.BlockSpec((BATCH, BL, D_MODEL), lambda i: (0, i, 0)),
        out_shape=jax.ShapeDtypeStruct((BATCH, MAX_LEN, D_MODEL), jnp.float32),
    )(inputs, pos_table, cond_emb)


def kernel(inputs, cond_pos, pos_table, cond_table):
    ctab_bf = cond_table.astype(jnp.bfloat16)
    ctab_pairs = jnp.stack([ctab_bf[:, :D_PACK], ctab_bf[:, D_PACK:]], axis=-1)
    ctab_rep = jnp.tile(lax.bitcast_convert_type(ctab_pairs, jnp.int32), (REP, 1))
    cond_emb = _sc_cond_emb(cond_pos, ctab_rep)
    return _dense_add(inputs, pos_table, cond_emb)


# speedup vs baseline: 1.0175x; 1.0175x over previous
import functools

import jax
import jax.numpy as jnp
from jax import lax
from jax.experimental import pallas as pl
from jax.experimental.pallas import tpu as pltpu
from jax.experimental.pallas import tpu_sc as plsc

MAX_LEN = 4096
D_MODEL = 768
BATCH = 4
NCOND = 3

NC, NS, NLANES = 2, 16, 16
NW = NC * NS
ROWS_PER_W = MAX_LEN // NW
CHUNK = 64
REP = 256
_sc_mesh = plsc.VectorSubcoreMesh(core_axis_name="c", subcore_axis_name="s")


D_PACK = D_MODEL // 2


@functools.partial(
    pl.kernel,
    out_type=jax.ShapeDtypeStruct((MAX_LEN, D_PACK), jnp.int32),
    mesh=_sc_mesh,
    scratch_types=[
        pltpu.VMEM((CHUNK,), jnp.int32),
        pltpu.VMEM((CHUNK,), jnp.int32),
        pltpu.VMEM((CHUNK, D_PACK), jnp.int32),
        pltpu.VMEM((CHUNK, D_PACK), jnp.int32),
        pltpu.SemaphoreType.DMA,
        pltpu.SemaphoreType.DMA,
    ],
)
def _sc_cond_emb(idx_hbm, ctab_hbm, cemb_hbm, idx0, idx1, buf0, buf1, sem0, sem1):
    wid = lax.axis_index("s") * NC + lax.axis_index("c")
    base = wid * ROWS_PER_W
    pltpu.sync_copy(idx_hbm.at[pl.ds(base, CHUNK)], idx0)
    pltpu.sync_copy(idx_hbm.at[pl.ds(base + CHUNK, CHUNK)], idx1)
    lanes = lax.broadcasted_iota(jnp.int32, (NLANES,), 0)
    for t in range(CHUNK // NLANES):
        o0 = ((base + t * NLANES) & (REP - 1)) + lanes
        idx0[pl.ds(t * NLANES, NLANES)] += NCOND * (o0 & (REP - 1))
        o1 = ((base + CHUNK + t * NLANES) & (REP - 1)) + lanes
        idx1[pl.ds(t * NLANES, NLANES)] += NCOND * (o1 & (REP - 1))
    cp0 = pltpu.async_copy(ctab_hbm.at[idx0], buf0, sem0)
    cp1 = pltpu.async_copy(ctab_hbm.at[idx1], buf1, sem1)
    cp0.wait()
    pltpu.sync_copy(buf0, cemb_hbm.at[pl.ds(base, CHUNK)])
    cp1.wait()
    pltpu.sync_copy(buf1, cemb_hbm.at[pl.ds(base + CHUNK, CHUNK)])


BL = 1024
NB = MAX_LEN // BL


def _dense_body(in_ref, pos_ref, cemb_ref, out_ref):
    x = cemb_ref[...]
    lo = lax.bitcast_convert_type(lax.shift_left(x, 16), jnp.float32)
    hi = lax.bitcast_convert_type(lax.bitwise_and(x, jnp.int32(-65536)), jnp.float32)
    add_lo = pos_ref[:, :D_PACK] + lo
    add_hi = pos_ref[:, D_PACK:] + hi
    out_ref[:, :, :D_PACK] = in_ref[:, :, :D_PACK] + add_lo[None, :, :]
    out_ref[:, :, D_PACK:] = in_ref[:, :, D_PACK:] + add_hi[None, :, :]


@jax.jit
def _dense_add(inputs, pos_table, cond_emb):
    return pl.pallas_call(
        _dense_body,
        grid=(NB,),
        in_specs=[
            pl.BlockSpec((BATCH, BL, D_MODEL), lambda i: (0, i, 0)),
            pl.BlockSpec((BL, D_MODEL), lambda i: (i, 0)),
            pl.BlockSpec((BL, D_PACK), lambda i: (i, 0)),
        ],
        out_specs=pl.BlockSpec((BATCH, BL, D_MODEL), lambda i: (0, i, 0)),
        out_shape=jax.ShapeDtypeStruct((BATCH, MAX_LEN, D_MODEL), jnp.float32),
    )(inputs, pos_table, cond_emb)


def kernel(inputs, cond_pos, pos_table, cond_table):
    ctab_bf = cond_table.astype(jnp.bfloat16)
    ctab_pairs = jnp.stack([ctab_bf[:, :D_PACK], ctab_bf[:, D_PACK:]], axis=-1)
    ctab_rep = jnp.tile(lax.bitcast_convert_type(ctab_pairs, jnp.int32), (REP, 1))
    cond_emb = _sc_cond_emb(cond_pos, ctab_rep)
    return _dense_add(inputs, pos_table, cond_emb)
